# baseline (speedup 1.0000x reference)
import functools

import jax
import jax.numpy as jnp
from jax import lax
from jax.experimental import pallas as pl
from jax.experimental.pallas import tpu as pltpu
from jax.experimental.pallas import tpu_sc as plsc

VOCAB = 1000000
EMBED = 64
HIDDEN = 128
CLASSES = 5
BATCH = 4096
SEQ = 200

CHUNK = 100
CPB = SEQ // CHUNK
NC = 2
NS = 16
NW = NC * NS
BPW = BATCH // NW
CPW = BPW * CPB
NV = EMBED // 16


def _pool_body(idx_hbm, table_hbm, out_hbm, idx_v, rows_v, acc_v, sem0, sem1):
    cid = lax.axis_index("c")
    sid = lax.axis_index("s")
    wid = sid * NC + cid
    cbase = wid * CPW
    obase = wid * BPW

    pltpu.sync_copy(idx_hbm.at[pl.ds(cbase, CPW)], idx_v)

    def start(c, buf, sem):
        pltpu.async_copy(table_hbm.at[idx_v.at[c]], rows_v.at[buf], sem)

    def wait(c, buf, sem):
        pltpu.make_async_copy(
            table_hbm.at[idx_v.at[c]], rows_v.at[buf], sem
        ).wait()

    def sum_chunk(buf, acc):
        def rbody(i, acc):
            accs = list(acc)
            for u in range(4):
                r = i * 4 + u
                for j in range(NV):
                    accs[j] = accs[j] + rows_v[buf, r, pl.ds(j * 16, 16)]
            return tuple(accs)

        return lax.fori_loop(0, CHUNK // 4, rbody, acc)

    start(0, 0, sem0)
    start(1, 1, sem1)

    def gbody(g, carry):
        zero = jnp.zeros((16,), jnp.float32)
        acc = (zero,) * NV
        wait(2 * g, 0, sem0)
        acc = sum_chunk(0, acc)

        @pl.when(g < BPW - 1)
        def _():
            start(2 * g + 2, 0, sem0)

        wait(2 * g + 1, 1, sem1)
        acc = sum_chunk(1, acc)

        @pl.when(g < BPW - 1)
        def _():
            start(2 * g + 3, 1, sem1)

        for j in range(NV):
            acc_v[g, pl.ds(j * 16, 16)] = acc[j]
        return carry

    lax.fori_loop(0, BPW, gbody, 0)
    pltpu.sync_copy(acc_v, out_hbm.at[pl.ds(obase, BPW)])


@functools.partial(jax.jit, static_argnames=())
def _sc_pool(idx2, table):
    mesh = plsc.VectorSubcoreMesh(core_axis_name="c", subcore_axis_name="s")
    return pl.kernel(
        _pool_body,
        out_type=jax.ShapeDtypeStruct((BATCH, EMBED), jnp.float32),
        mesh=mesh,
        scratch_types=[
            pltpu.VMEM((CPW, CHUNK), jnp.int32),
            pltpu.VMEM((2, CHUNK, EMBED), jnp.float32),
            pltpu.VMEM((BPW, EMBED), jnp.float32),
            pltpu.SemaphoreType.DMA,
            pltpu.SemaphoreType.DMA,
        ],
        compiler_params=pltpu.CompilerParams(use_tc_tiling_on_sc=False),
        name="cbow_pool_sc",
    )(idx2, table)


def _mlp_body(x_ref, w1_ref, b1_ref, w2_ref, b2_ref, w3_ref, b3_ref, o_ref):
    x = x_ref[...]
    h1 = jnp.tanh(
        jnp.dot(x, w1_ref[...], preferred_element_type=jnp.float32) + b1_ref[...]
    )
    h2 = jnp.tanh(
        jnp.dot(h1, w2_ref[...], preferred_element_type=jnp.float32) + b2_ref[...]
    )
    o_ref[...] = (
        jnp.dot(h2, w3_ref[...], preferred_element_type=jnp.float32) + b3_ref[...]
    )


def _tc_mlp(pooled, W1, b1, W2, b2, W3, b3):
    blk = 1024
    grid = (BATCH // blk,)
    full = lambda shape: pl.BlockSpec(shape, lambda i: (0,) * len(shape))
    return pl.pallas_call(
        _mlp_body,
        grid=grid,
        in_specs=[
            pl.BlockSpec((blk, EMBED), lambda i: (i, 0)),
            full((EMBED, HIDDEN)),
            full((1, HIDDEN)),
            full((HIDDEN, HIDDEN)),
            full((1, HIDDEN)),
            full((HIDDEN, CLASSES)),
            full((1, CLASSES)),
        ],
        out_specs=pl.BlockSpec((blk, CLASSES), lambda i: (i, 0)),
        out_shape=jax.ShapeDtypeStruct((BATCH, CLASSES), jnp.float32),
    )(pooled, W1, b1, W2, b2, W3, b3)


def kernel(inputs, embed_table, W1, b1, W2, b2, W3, b3):
    idx2 = inputs.reshape(BATCH * CPB, CHUNK)
    pooled = _sc_pool(idx2, embed_table)
    return _tc_mlp(
        pooled,
        W1,
        b1.reshape(1, HIDDEN),
        W2,
        b2.reshape(1, HIDDEN),
        W3,
        b3.reshape(1, CLASSES),
    )


# speedup vs baseline: 1.0034x; 1.0034x over previous
import functools

import jax
import jax.numpy as jnp
from jax import lax
from jax.experimental import pallas as pl
from jax.experimental.pallas import tpu as pltpu
from jax.experimental.pallas import tpu_sc as plsc

VOCAB = 1000000
EMBED = 64
HIDDEN = 128
CLASSES = 5
BATCH = 4096
SEQ = 200

CHUNKS = ((0, 104), (104, 96))
CHUNK = 104
NC = 2
NS = 16
NW = NC * NS
BPW = BATCH // NW
NV = EMBED // 16


def _pool_body(idx_hbm, table_hbm, out_hbm, idx_v, rows_v, acc_v, sem0, sem1):
    cid = lax.axis_index("c")
    sid = lax.axis_index("s")
    wid = sid * NC + cid
    obase = wid * BPW

    pltpu.sync_copy(idx_hbm.at[pl.ds(obase, BPW)], idx_v)

    def start(b, h, buf, sem):
        off, n = CHUNKS[h]
        pltpu.async_copy(
            table_hbm.at[idx_v.at[b, pl.ds(off, n)]],
            rows_v.at[buf, pl.ds(0, n)],
            sem,
        )

    def wait(b, h, buf, sem):
        off, n = CHUNKS[h]
        pltpu.make_async_copy(
            table_hbm.at[idx_v.at[b, pl.ds(off, n)]],
            rows_v.at[buf, pl.ds(0, n)],
            sem,
        ).wait()

    def sum_chunk(h, buf, acc):
        n = CHUNKS[h][1]

        def rbody(i, acc):
            accs = list(acc)
            for u in range(4):
                r = i * 4 + u
                for j in range(NV):
                    accs[j] = accs[j] + rows_v[buf, r, pl.ds(j * 16, 16)]
            return tuple(accs)

        return lax.fori_loop(0, n // 4, rbody, acc)

    start(0, 0, 0, sem0)
    start(0, 1, 1, sem1)

    def gbody(g, carry):
        zero = jnp.zeros((16,), jnp.float32)
        acc = (zero,) * NV
        wait(g, 0, 0, sem0)
        acc = sum_chunk(0, 0, acc)

        @pl.when(g < BPW - 1)
        def _():
            start(g + 1, 0, 0, sem0)

        wait(g, 1, 1, sem1)
        acc = sum_chunk(1, 1, acc)

        @pl.when(g < BPW - 1)
        def _():
            start(g + 1, 1, 1, sem1)

        for j in range(NV):
            acc_v[g, pl.ds(j * 16, 16)] = acc[j]
        return carry

    lax.fori_loop(0, BPW, gbody, 0)
    pltpu.sync_copy(acc_v, out_hbm.at[pl.ds(obase, BPW)])


@functools.partial(jax.jit, static_argnames=())
def _sc_pool(idx2, table):
    mesh = plsc.VectorSubcoreMesh(core_axis_name="c", subcore_axis_name="s")
    return pl.kernel(
        _pool_body,
        out_type=jax.ShapeDtypeStruct((BATCH, EMBED), jnp.float32),
        mesh=mesh,
        scratch_types=[
            pltpu.VMEM((BPW, SEQ), jnp.int32),
            pltpu.VMEM((2, CHUNK, EMBED), jnp.float32),
            pltpu.VMEM((BPW, EMBED), jnp.float32),
            pltpu.SemaphoreType.DMA,
            pltpu.SemaphoreType.DMA,
        ],
        compiler_params=pltpu.CompilerParams(use_tc_tiling_on_sc=False),
        name="cbow_pool_sc",
    )(idx2, table)


def _mlp_body(x_ref, w1_ref, b1_ref, w2_ref, b2_ref, w3_ref, b3_ref, o_ref):
    x = x_ref[...]
    h1 = jnp.tanh(
        jnp.dot(x, w1_ref[...], preferred_element_type=jnp.float32) + b1_ref[...]
    )
    h2 = jnp.tanh(
        jnp.dot(h1, w2_ref[...], preferred_element_type=jnp.float32) + b2_ref[...]
    )
    o_ref[...] = (
        jnp.dot(h2, w3_ref[...], preferred_element_type=jnp.float32) + b3_ref[...]
    )


def _tc_mlp(pooled, W1, b1, W2, b2, W3, b3):
    blk = 1024
    grid = (BATCH // blk,)
    full = lambda shape: pl.BlockSpec(shape, lambda i: (0,) * len(shape))
    return pl.pallas_call(
        _mlp_body,
        grid=grid,
        in_specs=[
            pl.BlockSpec((blk, EMBED), lambda i: (i, 0)),
            full((EMBED, HIDDEN)),
            full((1, HIDDEN)),
            full((HIDDEN, HIDDEN)),
            full((1, HIDDEN)),
            full((HIDDEN, CLASSES)),
            full((1, CLASSES)),
        ],
        out_specs=pl.BlockSpec((blk, CLASSES), lambda i: (i, 0)),
        out_shape=jax.ShapeDtypeStruct((BATCH, CLASSES), jnp.float32),
    )(pooled, W1, b1, W2, b2, W3, b3)


def kernel(inputs, embed_table, W1, b1, W2, b2, W3, b3):
    pooled = _sc_pool(inputs, embed_table)
    return _tc_mlp(
        pooled,
        W1,
        b1.reshape(1, HIDDEN),
        W2,
        b2.reshape(1, HIDDEN),
        W3,
        b3.reshape(1, CLASSES),
    )
